# baseline (speedup 1.0000x reference)
import jax
import jax.numpy as jnp
from jax import lax
from jax.experimental import pallas as pl
from jax.experimental.pallas import tpu as pltpu
from jax.experimental.pallas import tpu_sc as plsc

NUM_GRAPHS = 64
N = 10000
NP = 10240
NAGG = 10368
E = 640000
EP = 647168
CHUNK = 128
CHUNKS_PER_WORKER = EP // (32 * CHUNK)
CH0 = 190
CH1 = 126
DP = 128
VP = 512
IMIN = -2147483648
IMAX = 2147483647


def _segsum_body(vals_hbm, idx_hbm, zeros_hbm, out_hbm,
                 cb0, cb1, rows0, rows1, agg_sh, sem0, sem1):
    c = lax.axis_index("c")
    s = lax.axis_index("s")

    rows_per_tile = NAGG // 16
    pltpu.sync_copy(zeros_hbm.at[pl.ds(s * rows_per_tile, rows_per_tile)],
                    agg_sh.at[pl.ds(s * rows_per_tile, rows_per_tile)])
    plsc.subcore_barrier()

    base = jnp.where(c == 0, s * CH0, 16 * CH0 + s * CH1)
    npairs = jnp.where(c == 0, CH0 // 2, CH1 // 2)

    pltpu.sync_copy(idx_hbm.at[base], cb0)
    pltpu.sync_copy(idx_hbm.at[base + 1], cb1)
    pltpu.async_copy(vals_hbm.at[cb0.at[0, 0]], rows0, sem0)
    pltpu.async_copy(vals_hbm.at[cb1.at[0, 0]], rows1, sem1)

    def step(p, carry):
        j0 = base + 2 * p
        pltpu.make_async_copy(vals_hbm.at[cb0.at[0, 0]], rows0, sem0).wait()
        pltpu.sync_copy(rows0, agg_sh.at[cb0.at[1, 0]], add=True)

        @pl.when(p + 1 < npairs)
        def _():
            pltpu.sync_copy(idx_hbm.at[j0 + 2], cb0)
            pltpu.async_copy(vals_hbm.at[cb0.at[0, 0]], rows0, sem0)

        pltpu.make_async_copy(vals_hbm.at[cb1.at[0, 0]], rows1, sem1).wait()
        pltpu.sync_copy(rows1, agg_sh.at[cb1.at[1, 0]], add=True)

        @pl.when(p + 1 < npairs)
        def _():
            pltpu.sync_copy(idx_hbm.at[j0 + 3], cb1)
            pltpu.async_copy(vals_hbm.at[cb1.at[0, 0]], rows1, sem1)

        return carry

    lax.fori_loop(0, npairs, step, 0)
    plsc.subcore_barrier()

    out_rows = NP // 16
    pltpu.sync_copy(agg_sh.at[pl.ds(s * out_rows, out_rows)],
                    out_hbm.at[c, pl.ds(s * out_rows, out_rows)])


def _sc_segsum(vals, idx2d, zeros):
    mesh = plsc.VectorSubcoreMesh(core_axis_name="c", subcore_axis_name="s")
    f = pl.kernel(
        _segsum_body,
        out_type=jax.ShapeDtypeStruct((2, NP, DP), jnp.float32),
        mesh=mesh,
        scratch_types=[
            pltpu.VMEM((2, 1, CHUNK), jnp.int32),
            pltpu.VMEM((2, 1, CHUNK), jnp.int32),
            pltpu.VMEM((CHUNK, DP), jnp.float32),
            pltpu.VMEM((CHUNK, DP), jnp.float32),
            pltpu.VMEM_SHARED((NAGG, DP), jnp.float32),
            pltpu.SemaphoreType.DMA,
            pltpu.SemaphoreType.DMA,
        ],
    )
    return f(vals, idx2d, zeros)


def _prep_body(idx_ref, emb_ref, wr_ref, wroot_ref, u1_ref, root1_ref):
    emb_v = emb_ref[...]
    tab_rel = lax.dot_general(emb_v, wr_ref[...], (((1,), (1,)), ((), ())),
                              preferred_element_type=jnp.float32)
    tab_root = lax.dot_general(emb_v, wroot_ref[...], (((1,), (1,)), ((), ())),
                               preferred_element_type=jnp.float32)
    idx = idx_ref[...]
    oh = (idx == lax.broadcasted_iota(jnp.int32, (1, VP), 1))
    ohf = oh.astype(jnp.float32)
    u1_ref[...] = jnp.dot(ohf, tab_rel, preferred_element_type=jnp.float32)
    root1_ref[...] = jnp.dot(ohf, tab_root, preferred_element_type=jnp.float32)


def _prep(idx_pad, emb_pad, w1r_pad, w1root_pad):
    return pl.pallas_call(
        _prep_body,
        out_shape=(jax.ShapeDtypeStruct((NP, DP), jnp.float32),
                   jax.ShapeDtypeStruct((NP, DP), jnp.float32)),
    )(idx_pad, emb_pad, w1r_pad, w1root_pad)


def _topk_mask(score_col, batch_col, alive_col):
    bits = lax.bitcast_convert_type(score_col, jnp.int32)
    ki = jnp.where(bits >= 0, bits, jnp.int32(IMIN) - bits)
    ki = jnp.where(alive_col, ki, jnp.int32(IMIN))

    giota = lax.broadcasted_iota(jnp.int32, (1, NUM_GRAPHS), 1)
    galive = (batch_col == giota) & alive_col

    counts = jnp.sum(galive.astype(jnp.float32), axis=0, keepdims=True)
    k = jnp.ceil(jnp.float32(0.8) * counts).astype(jnp.int32)

    def bs_step(_, lohi):
        lo, hi = lohi
        mid = lo + lax.shift_right_logical(hi - lo, 1)
        cnt = jnp.sum((galive & (ki >= mid)).astype(jnp.int32),
                      axis=0, keepdims=True)
        ge = cnt >= k
        return (jnp.where(ge, mid, lo), jnp.where(ge, hi, mid))

    lo0 = jnp.full((1, NUM_GRAPHS), IMIN, jnp.int32)
    hi0 = jnp.full((1, NUM_GRAPHS), IMAX, jnp.int32)
    thr, _ = lax.fori_loop(0, 32, bs_step, (lo0, hi0))

    gt = galive & (ki > thr)
    cnt_gt = jnp.sum(gt.astype(jnp.int32), axis=0, keepdims=True)
    t_budget = k - cnt_gt
    tie = galive & (ki == thr)
    niota = lax.broadcasted_iota(jnp.int32, (NP, 1), 0)

    def idx_step(_, lohi):
        lo, hi = lohi
        mid = lo + lax.shift_right_logical(hi - lo, 1)
        cnt = jnp.sum((tie & (niota < mid)).astype(jnp.int32),
                      axis=0, keepdims=True)
        ge = cnt >= t_budget
        return (jnp.where(ge, lo, mid), jnp.where(ge, mid, hi))

    lo0i = jnp.zeros((1, NUM_GRAPHS), jnp.int32)
    hi0i = jnp.full((1, NUM_GRAPHS), 16384, jnp.int32)
    _, cut = lax.fori_loop(0, 14, idx_step, (lo0i, hi0i))

    keepMT = gt | (tie & (niota < cut) & (t_budget > 0))
    mask_col = jnp.sum(keepMT.astype(jnp.float32), axis=1, keepdims=True)
    return mask_col, keepMT, counts


def _pool_feats(x_new, keepMT):
    keepf = keepMT.astype(jnp.float32)
    counts_new = jnp.sum(keepf, axis=0, keepdims=True)
    sums = lax.dot_general(keepf, x_new, (((0,), (0,)), ((), ())),
                           preferred_element_type=jnp.float32)
    gap = sums / jnp.maximum(counts_new.reshape(NUM_GRAPHS, 1), 1.0)

    giota = lax.broadcasted_iota(jnp.int32, (1, NUM_GRAPHS), 1)
    giota_col = lax.broadcasted_iota(jnp.int32, (NUM_GRAPHS, 1), 0)

    def gmax(g, acc):
        colmask = jnp.sum(jnp.where(giota == g, keepf, 0.0),
                          axis=1, keepdims=True)
        big = jnp.where(colmask > 0, x_new, -jnp.inf)
        m = jnp.max(big, axis=0, keepdims=True)
        return jnp.where(giota_col == g, m, acc)

    gmp = lax.fori_loop(0, NUM_GRAPHS, gmax,
                        jnp.zeros((NUM_GRAPHS, DP), jnp.float32))
    gmp = jnp.where(gmp == -jnp.inf, 0.0, gmp)
    return jnp.concatenate([gmp, gap], axis=1)


def _block2_body(agg_ref, root1_ref, b1_ref, batch_ref, p1_ref,
                 g_ref, x1_ref, mask1_ref):
    agg = agg_ref[0] + agg_ref[1]
    h1 = jnp.maximum(agg + b1_ref[...] + root1_ref[...], 0.0)
    p1 = p1_ref[...]
    score = jnp.dot(h1, p1, preferred_element_type=jnp.float32)
    score = score * lax.rsqrt(jnp.sum(p1 * p1))
    batch = batch_ref[...]
    alive = batch < NUM_GRAPHS
    mask, keepMT, _ = _topk_mask(score, batch, alive)
    g_arr = h1 * (jnp.tanh(score) * mask)
    x1_ref[...] = _pool_feats(g_arr, keepMT)
    g_ref[...] = g_arr
    mask1_ref[...] = mask


def _block2(agg1, root1, b1_pad, batch_pad, p1_pad):
    return pl.pallas_call(
        _block2_body,
        out_shape=(jax.ShapeDtypeStruct((NP, DP), jnp.float32),
                   jax.ShapeDtypeStruct((NUM_GRAPHS, 2 * DP), jnp.float32),
                   jax.ShapeDtypeStruct((NP, 1), jnp.float32)),
    )(agg1, root1, b1_pad, batch_pad, p1_pad)


def _proj_body(g_ref, w_ref, y_ref):
    y_ref[...] = lax.dot_general(g_ref[...], w_ref[...], (((1,), (1,)), ((), ())),
                                 preferred_element_type=jnp.float32)


def _proj(g_arr, w_pad):
    return pl.pallas_call(
        _proj_body,
        out_shape=jax.ShapeDtypeStruct((NP, DP), jnp.float32),
    )(g_arr, w_pad)


def _block3_body(agg_ref, g_ref, w2root_ref, b2_ref, batch_ref, mask1_ref,
                 p2_ref, x1_ref, wl1_ref, bl1_ref, wl3_ref, bl3_ref, out_ref):
    agg = agg_ref[0] + agg_ref[1]
    mask1 = mask1_ref[...]
    root2 = lax.dot_general(g_ref[...], w2root_ref[...],
                            (((1,), (1,)), ((), ())),
                            preferred_element_type=jnp.float32)
    h2 = jnp.maximum(agg + b2_ref[...] + root2, 0.0) * mask1
    p2 = p2_ref[...]
    score = jnp.dot(h2, p2, preferred_element_type=jnp.float32)
    score = score * lax.rsqrt(jnp.sum(p2 * p2))
    batch = batch_ref[...]
    alive = mask1 > 0
    mask2, keepMT, _ = _topk_mask(score, batch, alive)
    x_new = h2 * (jnp.tanh(score) * mask2)
    x2 = _pool_feats(x_new, keepMT)
    feats = x1_ref[...] + x2
    hid = lax.dot_general(feats, wl1_ref[...], (((1,), (1,)), ((), ())),
                          preferred_element_type=jnp.float32)
    hid = jnp.maximum(hid + bl1_ref[...], 0.0)
    o = lax.dot_general(hid, wl3_ref[...], (((1,), (1,)), ((), ())),
                        preferred_element_type=jnp.float32) + bl3_ref[...]
    out_ref[...] = 1.0 / (1.0 + jnp.exp(-o))


def _block3(agg2, g_arr, w2root_pad, b2_pad, batch_pad, mask1, p2_pad,
            x1, wl1_pad, bl1_pad, wl3_pad, bl3_pad):
    return pl.pallas_call(
        _block3_body,
        out_shape=jax.ShapeDtypeStruct((NUM_GRAPHS, 128), jnp.float32),
    )(agg2, g_arr, w2root_pad, b2_pad, batch_pad, mask1, p2_pad,
      x1, wl1_pad, bl1_pad, wl3_pad, bl3_pad)


def _pad2(a, r, c):
    return jnp.pad(a, ((0, r - a.shape[0]), (0, c - a.shape[1])))


def kernel(x, edge_index, batch, emb, W1r, b1r, W1root, p1, W2r, b2r,
           W2root, p2, Wl1, bl1, Wl3, bl3):
    idx = x.astype(jnp.int32)
    idx_pad = jnp.pad(idx, ((0, NP - N), (0, 0)), constant_values=VP - 1)
    batch_pad = jnp.pad(batch.astype(jnp.int32)[:, None], ((0, NP - N), (0, 0)),
                        constant_values=NUM_GRAPHS)

    emb_pad = _pad2(emb, VP - 1, DP)
    emb_pad = jnp.pad(emb_pad, ((0, 1), (0, 0)))
    w1r_pad = _pad2(W1r, DP, DP)
    w1root_pad = _pad2(W1root, DP, DP)
    w2r_pad = _pad2(W2r, DP, DP)
    w2root_pad = _pad2(W2root, DP, DP)
    b1_pad = jnp.pad(b1r, (0, DP - b1r.shape[0]))[None, :]
    b2_pad = jnp.pad(b2r, (0, DP - b2r.shape[0]))[None, :]
    p1_pad = jnp.pad(p1, (0, DP - p1.shape[0]))[:, None]
    p2_pad = jnp.pad(p2, (0, DP - p2.shape[0]))[:, None]
    wl1_pad = _pad2(Wl1, 64, 2 * DP)
    wl1_pad = wl1_pad.at[:, DP:DP + 117].set(wl1_pad[:, 117:234])
    wl1_pad = wl1_pad.at[:, 117:DP].set(0.0)
    bl1_pad = jnp.pad(bl1, (0, 64 - bl1.shape[0]))[None, :]
    wl3_pad = _pad2(Wl3, 128, 64)
    bl3_pad = jnp.pad(bl3, (0, 128 - bl3.shape[0]))[None, :]

    srcp = jnp.pad(edge_index[0], (0, EP - E)).reshape(EP // CHUNK, 1, CHUNK)
    dstp = jnp.pad(edge_index[1], (0, EP - E),
                   constant_values=NP - 1).reshape(EP // CHUNK, 1, CHUNK)
    idx2d = jnp.concatenate([srcp[:, :, None, :], dstp[:, :, None, :]],
                            axis=1)
    zeros = jnp.zeros((NAGG, DP), jnp.float32)

    u1, root1 = _prep(idx_pad, emb_pad, w1r_pad, w1root_pad)
    agg1 = _sc_segsum(u1, idx2d, zeros)
    g_arr, x1, mask1 = _block2(agg1, root1, b1_pad, batch_pad, p1_pad)
    y2 = _proj(g_arr, w2r_pad)
    agg2 = _sc_segsum(y2, idx2d, zeros)
    out = _block3(agg2, g_arr, w2root_pad, b2_pad, batch_pad, mask1, p2_pad,
                  x1, wl1_pad, bl1_pad, wl3_pad, bl3_pad)
    return out[:, 0]


# speedup vs baseline: 1.0597x; 1.0597x over previous
import jax
import jax.numpy as jnp
from jax import lax
from jax.experimental import pallas as pl
from jax.experimental.pallas import tpu as pltpu
from jax.experimental.pallas import tpu_sc as plsc

NUM_GRAPHS = 64
N = 10000
NP = 10240
NAGG = 10368
E = 640000
EP = 647168
CHUNK = 128
CHUNKS_PER_WORKER = EP // (32 * CHUNK)
CH0 = 230
CH1 = 86
DP = 128
VP = 512
IMIN = -2147483648
IMAX = 2147483647


def _segsum_body(vals_hbm, idx_hbm, zeros_hbm, out_hbm,
                 cb0, cb1, rows0, rows1, agg_sh, sem0, sem1):
    c = lax.axis_index("c")
    s = lax.axis_index("s")

    rows_per_tile = NAGG // 16
    pltpu.sync_copy(zeros_hbm.at[pl.ds(s * rows_per_tile, rows_per_tile)],
                    agg_sh.at[pl.ds(s * rows_per_tile, rows_per_tile)])
    plsc.subcore_barrier()

    base = jnp.where(c == 0, s * CH0, 16 * CH0 + s * CH1)
    npairs = jnp.where(c == 0, CH0 // 2, CH1 // 2)

    pltpu.sync_copy(idx_hbm.at[base], cb0)
    pltpu.sync_copy(idx_hbm.at[base + 1], cb1)
    pltpu.async_copy(vals_hbm.at[cb0.at[0, 0]], rows0, sem0)
    pltpu.async_copy(vals_hbm.at[cb1.at[0, 0]], rows1, sem1)

    def step(p, carry):
        j0 = base + 2 * p
        pltpu.make_async_copy(vals_hbm.at[cb0.at[0, 0]], rows0, sem0).wait()
        pltpu.sync_copy(rows0, agg_sh.at[cb0.at[1, 0]], add=True)

        @pl.when(p + 1 < npairs)
        def _():
            pltpu.sync_copy(idx_hbm.at[j0 + 2], cb0)
            pltpu.async_copy(vals_hbm.at[cb0.at[0, 0]], rows0, sem0)

        pltpu.make_async_copy(vals_hbm.at[cb1.at[0, 0]], rows1, sem1).wait()
        pltpu.sync_copy(rows1, agg_sh.at[cb1.at[1, 0]], add=True)

        @pl.when(p + 1 < npairs)
        def _():
            pltpu.sync_copy(idx_hbm.at[j0 + 3], cb1)
            pltpu.async_copy(vals_hbm.at[cb1.at[0, 0]], rows1, sem1)

        return carry

    lax.fori_loop(0, npairs, step, 0)
    plsc.subcore_barrier()

    out_rows = NP // 16
    pltpu.sync_copy(agg_sh.at[pl.ds(s * out_rows, out_rows)],
                    out_hbm.at[c, pl.ds(s * out_rows, out_rows)])


def _sc_segsum(vals, idx2d, zeros):
    mesh = plsc.VectorSubcoreMesh(core_axis_name="c", subcore_axis_name="s")
    f = pl.kernel(
        _segsum_body,
        out_type=jax.ShapeDtypeStruct((2, NP, DP), jnp.float32),
        mesh=mesh,
        scratch_types=[
            pltpu.VMEM((2, 1, CHUNK), jnp.int32),
            pltpu.VMEM((2, 1, CHUNK), jnp.int32),
            pltpu.VMEM((CHUNK, DP), jnp.float32),
            pltpu.VMEM((CHUNK, DP), jnp.float32),
            pltpu.VMEM_SHARED((NAGG, DP), jnp.float32),
            pltpu.SemaphoreType.DMA,
            pltpu.SemaphoreType.DMA,
        ],
    )
    return f(vals, idx2d, zeros)


def _prep_body(idx_ref, emb_ref, wr_ref, wroot_ref, u1_ref, root1_ref):
    emb_v = emb_ref[...]
    tab_rel = lax.dot_general(emb_v, wr_ref[...], (((1,), (1,)), ((), ())),
                              preferred_element_type=jnp.float32)
    tab_root = lax.dot_general(emb_v, wroot_ref[...], (((1,), (1,)), ((), ())),
                               preferred_element_type=jnp.float32)
    idx = idx_ref[...]
    oh = (idx == lax.broadcasted_iota(jnp.int32, (1, VP), 1))
    ohf = oh.astype(jnp.float32)
    u1_ref[...] = jnp.dot(ohf, tab_rel, preferred_element_type=jnp.float32)
    root1_ref[...] = jnp.dot(ohf, tab_root, preferred_element_type=jnp.float32)


def _prep(idx_pad, emb_pad, w1r_pad, w1root_pad):
    return pl.pallas_call(
        _prep_body,
        out_shape=(jax.ShapeDtypeStruct((NP, DP), jnp.float32),
                   jax.ShapeDtypeStruct((NP, DP), jnp.float32)),
    )(idx_pad, emb_pad, w1r_pad, w1root_pad)


def _topk_mask(score_col, batch_col, alive_col):
    bits = lax.bitcast_convert_type(score_col, jnp.int32)
    ki = jnp.where(bits >= 0, bits, jnp.int32(IMIN) - bits)
    ki = jnp.where(alive_col, ki, jnp.int32(IMIN))

    giota = lax.broadcasted_iota(jnp.int32, (1, NUM_GRAPHS), 1)
    galive = (batch_col == giota) & alive_col

    counts = jnp.sum(galive.astype(jnp.float32), axis=0, keepdims=True)
    k = jnp.ceil(jnp.float32(0.8) * counts).astype(jnp.int32)

    def bs_step(_, lohi):
        lo, hi = lohi
        mid = lo + lax.shift_right_logical(hi - lo, 1)
        cnt = jnp.sum((galive & (ki >= mid)).astype(jnp.int32),
                      axis=0, keepdims=True)
        ge = cnt >= k
        return (jnp.where(ge, mid, lo), jnp.where(ge, hi, mid))

    lo0 = jnp.full((1, NUM_GRAPHS), IMIN, jnp.int32)
    hi0 = jnp.full((1, NUM_GRAPHS), IMAX, jnp.int32)
    thr, _ = lax.fori_loop(0, 32, bs_step, (lo0, hi0))

    gt = galive & (ki > thr)
    cnt_gt = jnp.sum(gt.astype(jnp.int32), axis=0, keepdims=True)
    t_budget = k - cnt_gt
    tie = galive & (ki == thr)
    niota = lax.broadcasted_iota(jnp.int32, (NP, 1), 0)

    def idx_step(_, lohi):
        lo, hi = lohi
        mid = lo + lax.shift_right_logical(hi - lo, 1)
        cnt = jnp.sum((tie & (niota < mid)).astype(jnp.int32),
                      axis=0, keepdims=True)
        ge = cnt >= t_budget
        return (jnp.where(ge, lo, mid), jnp.where(ge, mid, hi))

    lo0i = jnp.zeros((1, NUM_GRAPHS), jnp.int32)
    hi0i = jnp.full((1, NUM_GRAPHS), 16384, jnp.int32)
    _, cut = lax.fori_loop(0, 14, idx_step, (lo0i, hi0i))

    keepMT = gt | (tie & (niota < cut) & (t_budget > 0))
    mask_col = jnp.sum(keepMT.astype(jnp.float32), axis=1, keepdims=True)
    return mask_col, keepMT, counts


def _pool_feats(x_new, keepMT):
    keepf = keepMT.astype(jnp.float32)
    counts_new = jnp.sum(keepf, axis=0, keepdims=True)
    sums = lax.dot_general(keepf, x_new, (((0,), (0,)), ((), ())),
                           preferred_element_type=jnp.float32)
    gap = sums / jnp.maximum(counts_new.reshape(NUM_GRAPHS, 1), 1.0)

    giota = lax.broadcasted_iota(jnp.int32, (1, NUM_GRAPHS), 1)
    giota_col = lax.broadcasted_iota(jnp.int32, (NUM_GRAPHS, 1), 0)

    def gmax(g, acc):
        colmask = jnp.sum(jnp.where(giota == g, keepf, 0.0),
                          axis=1, keepdims=True)
        big = jnp.where(colmask > 0, x_new, -jnp.inf)
        m = jnp.max(big, axis=0, keepdims=True)
        return jnp.where(giota_col == g, m, acc)

    gmp = lax.fori_loop(0, NUM_GRAPHS, gmax,
                        jnp.zeros((NUM_GRAPHS, DP), jnp.float32))
    gmp = jnp.where(gmp == -jnp.inf, 0.0, gmp)
    return jnp.concatenate([gmp, gap], axis=1)


def _block2_body(agg_ref, root1_ref, b1_ref, batch_ref, p1_ref,
                 g_ref, x1_ref, mask1_ref):
    agg = agg_ref[0] + agg_ref[1]
    h1 = jnp.maximum(agg + b1_ref[...] + root1_ref[...], 0.0)
    p1 = p1_ref[...]
    score = jnp.dot(h1, p1, preferred_element_type=jnp.float32)
    score = score * lax.rsqrt(jnp.sum(p1 * p1))
    batch = batch_ref[...]
    alive = batch < NUM_GRAPHS
    mask, keepMT, _ = _topk_mask(score, batch, alive)
    g_arr = h1 * (jnp.tanh(score) * mask)
    x1_ref[...] = _pool_feats(g_arr, keepMT)
    g_ref[...] = g_arr
    mask1_ref[...] = mask


def _block2(agg1, root1, b1_pad, batch_pad, p1_pad):
    return pl.pallas_call(
        _block2_body,
        out_shape=(jax.ShapeDtypeStruct((NP, DP), jnp.float32),
                   jax.ShapeDtypeStruct((NUM_GRAPHS, 2 * DP), jnp.float32),
                   jax.ShapeDtypeStruct((NP, 1), jnp.float32)),
    )(agg1, root1, b1_pad, batch_pad, p1_pad)


def _proj_body(g_ref, w_ref, y_ref):
    y_ref[...] = lax.dot_general(g_ref[...], w_ref[...], (((1,), (1,)), ((), ())),
                                 preferred_element_type=jnp.float32)


def _proj(g_arr, w_pad):
    return pl.pallas_call(
        _proj_body,
        out_shape=jax.ShapeDtypeStruct((NP, DP), jnp.float32),
    )(g_arr, w_pad)


def _block3_body(agg_ref, g_ref, w2root_ref, b2_ref, batch_ref, mask1_ref,
                 p2_ref, x1_ref, wl1_ref, bl1_ref, wl3_ref, bl3_ref, out_ref):
    agg = agg_ref[0] + agg_ref[1]
    mask1 = mask1_ref[...]
    root2 = lax.dot_general(g_ref[...], w2root_ref[...],
                            (((1,), (1,)), ((), ())),
                            preferred_element_type=jnp.float32)
    h2 = jnp.maximum(agg + b2_ref[...] + root2, 0.0) * mask1
    p2 = p2_ref[...]
    score = jnp.dot(h2, p2, preferred_element_type=jnp.float32)
    score = score * lax.rsqrt(jnp.sum(p2 * p2))
    batch = batch_ref[...]
    alive = mask1 > 0
    mask2, keepMT, _ = _topk_mask(score, batch, alive)
    x_new = h2 * (jnp.tanh(score) * mask2)
    x2 = _pool_feats(x_new, keepMT)
    feats = x1_ref[...] + x2
    hid = lax.dot_general(feats, wl1_ref[...], (((1,), (1,)), ((), ())),
                          preferred_element_type=jnp.float32)
    hid = jnp.maximum(hid + bl1_ref[...], 0.0)
    o = lax.dot_general(hid, wl3_ref[...], (((1,), (1,)), ((), ())),
                        preferred_element_type=jnp.float32) + bl3_ref[...]
    out_ref[...] = 1.0 / (1.0 + jnp.exp(-o))


def _block3(agg2, g_arr, w2root_pad, b2_pad, batch_pad, mask1, p2_pad,
            x1, wl1_pad, bl1_pad, wl3_pad, bl3_pad):
    return pl.pallas_call(
        _block3_body,
        out_shape=jax.ShapeDtypeStruct((NUM_GRAPHS, 128), jnp.float32),
    )(agg2, g_arr, w2root_pad, b2_pad, batch_pad, mask1, p2_pad,
      x1, wl1_pad, bl1_pad, wl3_pad, bl3_pad)


def _pad2(a, r, c):
    return jnp.pad(a, ((0, r - a.shape[0]), (0, c - a.shape[1])))


def kernel(x, edge_index, batch, emb, W1r, b1r, W1root, p1, W2r, b2r,
           W2root, p2, Wl1, bl1, Wl3, bl3):
    idx = x.astype(jnp.int32)
    idx_pad = jnp.pad(idx, ((0, NP - N), (0, 0)), constant_values=VP - 1)
    batch_pad = jnp.pad(batch.astype(jnp.int32)[:, None], ((0, NP - N), (0, 0)),
                        constant_values=NUM_GRAPHS)

    emb_pad = _pad2(emb, VP - 1, DP)
    emb_pad = jnp.pad(emb_pad, ((0, 1), (0, 0)))
    w1r_pad = _pad2(W1r, DP, DP)
    w1root_pad = _pad2(W1root, DP, DP)
    w2r_pad = _pad2(W2r, DP, DP)
    w2root_pad = _pad2(W2root, DP, DP)
    b1_pad = jnp.pad(b1r, (0, DP - b1r.shape[0]))[None, :]
    b2_pad = jnp.pad(b2r, (0, DP - b2r.shape[0]))[None, :]
    p1_pad = jnp.pad(p1, (0, DP - p1.shape[0]))[:, None]
    p2_pad = jnp.pad(p2, (0, DP - p2.shape[0]))[:, None]
    wl1_pad = _pad2(Wl1, 64, 2 * DP)
    wl1_pad = wl1_pad.at[:, DP:DP + 117].set(wl1_pad[:, 117:234])
    wl1_pad = wl1_pad.at[:, 117:DP].set(0.0)
    bl1_pad = jnp.pad(bl1, (0, 64 - bl1.shape[0]))[None, :]
    wl3_pad = _pad2(Wl3, 128, 64)
    bl3_pad = jnp.pad(bl3, (0, 128 - bl3.shape[0]))[None, :]

    srcp = jnp.pad(edge_index[0], (0, EP - E)).reshape(EP // CHUNK, 1, CHUNK)
    dstp = jnp.pad(edge_index[1], (0, EP - E),
                   constant_values=NP - 1).reshape(EP // CHUNK, 1, CHUNK)
    idx2d = jnp.concatenate([srcp[:, :, None, :], dstp[:, :, None, :]],
                            axis=1)
    zeros = jnp.zeros((NAGG, DP), jnp.float32)

    u1, root1 = _prep(idx_pad, emb_pad, w1r_pad, w1root_pad)
    agg1 = _sc_segsum(u1, idx2d, zeros)
    g_arr, x1, mask1 = _block2(agg1, root1, b1_pad, batch_pad, p1_pad)
    y2 = _proj(g_arr, w2r_pad)
    agg2 = _sc_segsum(y2, idx2d, zeros)
    out = _block3(agg2, g_arr, w2root_pad, b2_pad, batch_pad, mask1, p2_pad,
                  x1, wl1_pad, bl1_pad, wl3_pad, bl3_pad)
    return out[:, 0]


# speedup vs baseline: 1.0874x; 1.0261x over previous
import jax
import jax.numpy as jnp
from jax import lax
from jax.experimental import pallas as pl
from jax.experimental.pallas import tpu as pltpu
from jax.experimental.pallas import tpu_sc as plsc

NUM_GRAPHS = 64
N = 10000
NP = 10240
NAGG = 10368
E = 640000
EP = 647168
CHUNK = 128
CHUNKS_PER_WORKER = EP // (32 * CHUNK)
CH0 = 246
CH1 = 70
DP = 128
VP = 512
IMIN = -2147483648
IMAX = 2147483647


def _segsum_body(vals_hbm, idx_hbm, zeros_hbm, out_hbm,
                 cb0, cb1, rows0, rows1, agg_sh, sem0, sem1):
    c = lax.axis_index("c")
    s = lax.axis_index("s")

    rows_per_tile = NAGG // 16
    pltpu.sync_copy(zeros_hbm.at[pl.ds(s * rows_per_tile, rows_per_tile)],
                    agg_sh.at[pl.ds(s * rows_per_tile, rows_per_tile)])
    plsc.subcore_barrier()

    base = jnp.where(c == 0, s * CH0, 16 * CH0 + s * CH1)
    npairs = jnp.where(c == 0, CH0 // 2, CH1 // 2)

    pltpu.sync_copy(idx_hbm.at[base], cb0)
    pltpu.sync_copy(idx_hbm.at[base + 1], cb1)
    pltpu.async_copy(vals_hbm.at[cb0.at[0, 0]], rows0, sem0)
    pltpu.async_copy(vals_hbm.at[cb1.at[0, 0]], rows1, sem1)

    def step(p, carry):
        j0 = base + 2 * p
        pltpu.make_async_copy(vals_hbm.at[cb0.at[0, 0]], rows0, sem0).wait()
        pltpu.sync_copy(rows0, agg_sh.at[cb0.at[1, 0]], add=True)

        @pl.when(p + 1 < npairs)
        def _():
            pltpu.sync_copy(idx_hbm.at[j0 + 2], cb0)
            pltpu.async_copy(vals_hbm.at[cb0.at[0, 0]], rows0, sem0)

        pltpu.make_async_copy(vals_hbm.at[cb1.at[0, 0]], rows1, sem1).wait()
        pltpu.sync_copy(rows1, agg_sh.at[cb1.at[1, 0]], add=True)

        @pl.when(p + 1 < npairs)
        def _():
            pltpu.sync_copy(idx_hbm.at[j0 + 3], cb1)
            pltpu.async_copy(vals_hbm.at[cb1.at[0, 0]], rows1, sem1)

        return carry

    lax.fori_loop(0, npairs, step, 0)
    plsc.subcore_barrier()

    out_rows = NP // 16
    pltpu.sync_copy(agg_sh.at[pl.ds(s * out_rows, out_rows)],
                    out_hbm.at[c, pl.ds(s * out_rows, out_rows)])


def _sc_segsum(vals, idx2d, zeros):
    mesh = plsc.VectorSubcoreMesh(core_axis_name="c", subcore_axis_name="s")
    f = pl.kernel(
        _segsum_body,
        out_type=jax.ShapeDtypeStruct((2, NP, DP), jnp.float32),
        mesh=mesh,
        scratch_types=[
            pltpu.VMEM((2, 1, CHUNK), jnp.int32),
            pltpu.VMEM((2, 1, CHUNK), jnp.int32),
            pltpu.VMEM((CHUNK, DP), jnp.float32),
            pltpu.VMEM((CHUNK, DP), jnp.float32),
            pltpu.VMEM_SHARED((NAGG, DP), jnp.float32),
            pltpu.SemaphoreType.DMA,
            pltpu.SemaphoreType.DMA,
        ],
    )
    return f(vals, idx2d, zeros)


def _prep_body(idx_ref, emb_ref, wr_ref, wroot_ref, u1_ref, root1_ref):
    emb_v = emb_ref[...]
    tab_rel = lax.dot_general(emb_v, wr_ref[...], (((1,), (1,)), ((), ())),
                              preferred_element_type=jnp.float32)
    tab_root = lax.dot_general(emb_v, wroot_ref[...], (((1,), (1,)), ((), ())),
                               preferred_element_type=jnp.float32)
    idx = idx_ref[...]
    oh = (idx == lax.broadcasted_iota(jnp.int32, (1, VP), 1))
    ohf = oh.astype(jnp.float32)
    u1_ref[...] = jnp.dot(ohf, tab_rel, preferred_element_type=jnp.float32)
    root1_ref[...] = jnp.dot(ohf, tab_root, preferred_element_type=jnp.float32)


def _prep(idx_pad, emb_pad, w1r_pad, w1root_pad):
    return pl.pallas_call(
        _prep_body,
        out_shape=(jax.ShapeDtypeStruct((NP, DP), jnp.float32),
                   jax.ShapeDtypeStruct((NP, DP), jnp.float32)),
    )(idx_pad, emb_pad, w1r_pad, w1root_pad)


def _topk_mask(score_col, batch_col, alive_col):
    bits = lax.bitcast_convert_type(score_col, jnp.int32)
    ki = jnp.where(bits >= 0, bits, jnp.int32(IMIN) - bits)
    ki = jnp.where(alive_col, ki, jnp.int32(IMIN))

    giota = lax.broadcasted_iota(jnp.int32, (1, NUM_GRAPHS), 1)
    galive = (batch_col == giota) & alive_col

    counts = jnp.sum(galive.astype(jnp.float32), axis=0, keepdims=True)
    k = jnp.ceil(jnp.float32(0.8) * counts).astype(jnp.int32)

    def bs_step(_, lohi):
        lo, hi = lohi
        mid = lo + lax.shift_right_logical(hi - lo, 1)
        cnt = jnp.sum((galive & (ki >= mid)).astype(jnp.int32),
                      axis=0, keepdims=True)
        ge = cnt >= k
        return (jnp.where(ge, mid, lo), jnp.where(ge, hi, mid))

    lo0 = jnp.full((1, NUM_GRAPHS), IMIN, jnp.int32)
    hi0 = jnp.full((1, NUM_GRAPHS), IMAX, jnp.int32)
    thr, _ = lax.fori_loop(0, 32, bs_step, (lo0, hi0))

    gt = galive & (ki > thr)
    cnt_gt = jnp.sum(gt.astype(jnp.int32), axis=0, keepdims=True)
    t_budget = k - cnt_gt
    tie = galive & (ki == thr)
    niota = lax.broadcasted_iota(jnp.int32, (NP, 1), 0)

    def idx_step(_, lohi):
        lo, hi = lohi
        mid = lo + lax.shift_right_logical(hi - lo, 1)
        cnt = jnp.sum((tie & (niota < mid)).astype(jnp.int32),
                      axis=0, keepdims=True)
        ge = cnt >= t_budget
        return (jnp.where(ge, lo, mid), jnp.where(ge, mid, hi))

    lo0i = jnp.zeros((1, NUM_GRAPHS), jnp.int32)
    hi0i = jnp.full((1, NUM_GRAPHS), 16384, jnp.int32)
    _, cut = lax.fori_loop(0, 14, idx_step, (lo0i, hi0i))

    keepMT = gt | (tie & (niota < cut) & (t_budget > 0))
    mask_col = jnp.sum(keepMT.astype(jnp.float32), axis=1, keepdims=True)
    return mask_col, keepMT, counts


def _pool_feats(x_new, keepMT):
    keepf = keepMT.astype(jnp.float32)
    counts_new = jnp.sum(keepf, axis=0, keepdims=True)
    sums = lax.dot_general(keepf, x_new, (((0,), (0,)), ((), ())),
                           preferred_element_type=jnp.float32)
    gap = sums / jnp.maximum(counts_new.reshape(NUM_GRAPHS, 1), 1.0)

    giota = lax.broadcasted_iota(jnp.int32, (1, NUM_GRAPHS), 1)
    giota_col = lax.broadcasted_iota(jnp.int32, (NUM_GRAPHS, 1), 0)

    def gmax(g, acc):
        colmask = jnp.sum(jnp.where(giota == g, keepf, 0.0),
                          axis=1, keepdims=True)
        big = jnp.where(colmask > 0, x_new, -jnp.inf)
        m = jnp.max(big, axis=0, keepdims=True)
        return jnp.where(giota_col == g, m, acc)

    gmp = lax.fori_loop(0, NUM_GRAPHS, gmax,
                        jnp.zeros((NUM_GRAPHS, DP), jnp.float32))
    gmp = jnp.where(gmp == -jnp.inf, 0.0, gmp)
    return jnp.concatenate([gmp, gap], axis=1)


def _block2_body(agg_ref, root1_ref, b1_ref, batch_ref, p1_ref,
                 g_ref, x1_ref, mask1_ref):
    agg = agg_ref[0] + agg_ref[1]
    h1 = jnp.maximum(agg + b1_ref[...] + root1_ref[...], 0.0)
    p1 = p1_ref[...]
    score = jnp.dot(h1, p1, preferred_element_type=jnp.float32)
    score = score * lax.rsqrt(jnp.sum(p1 * p1))
    batch = batch_ref[...]
    alive = batch < NUM_GRAPHS
    mask, keepMT, _ = _topk_mask(score, batch, alive)
    g_arr = h1 * (jnp.tanh(score) * mask)
    x1_ref[...] = _pool_feats(g_arr, keepMT)
    g_ref[...] = g_arr
    mask1_ref[...] = mask


def _block2(agg1, root1, b1_pad, batch_pad, p1_pad):
    return pl.pallas_call(
        _block2_body,
        out_shape=(jax.ShapeDtypeStruct((NP, DP), jnp.float32),
                   jax.ShapeDtypeStruct((NUM_GRAPHS, 2 * DP), jnp.float32),
                   jax.ShapeDtypeStruct((NP, 1), jnp.float32)),
    )(agg1, root1, b1_pad, batch_pad, p1_pad)


def _proj_body(g_ref, w_ref, y_ref):
    y_ref[...] = lax.dot_general(g_ref[...], w_ref[...], (((1,), (1,)), ((), ())),
                                 preferred_element_type=jnp.float32)


def _proj(g_arr, w_pad):
    return pl.pallas_call(
        _proj_body,
        out_shape=jax.ShapeDtypeStruct((NP, DP), jnp.float32),
    )(g_arr, w_pad)


def _block3_body(agg_ref, g_ref, w2root_ref, b2_ref, batch_ref, mask1_ref,
                 p2_ref, x1_ref, wl1_ref, bl1_ref, wl3_ref, bl3_ref, out_ref):
    agg = agg_ref[0] + agg_ref[1]
    mask1 = mask1_ref[...]
    root2 = lax.dot_general(g_ref[...], w2root_ref[...],
                            (((1,), (1,)), ((), ())),
                            preferred_element_type=jnp.float32)
    h2 = jnp.maximum(agg + b2_ref[...] + root2, 0.0) * mask1
    p2 = p2_ref[...]
    score = jnp.dot(h2, p2, preferred_element_type=jnp.float32)
    score = score * lax.rsqrt(jnp.sum(p2 * p2))
    batch = batch_ref[...]
    alive = mask1 > 0
    mask2, keepMT, _ = _topk_mask(score, batch, alive)
    x_new = h2 * (jnp.tanh(score) * mask2)
    x2 = _pool_feats(x_new, keepMT)
    feats = x1_ref[...] + x2
    hid = lax.dot_general(feats, wl1_ref[...], (((1,), (1,)), ((), ())),
                          preferred_element_type=jnp.float32)
    hid = jnp.maximum(hid + bl1_ref[...], 0.0)
    o = lax.dot_general(hid, wl3_ref[...], (((1,), (1,)), ((), ())),
                        preferred_element_type=jnp.float32) + bl3_ref[...]
    out_ref[...] = 1.0 / (1.0 + jnp.exp(-o))


def _block3(agg2, g_arr, w2root_pad, b2_pad, batch_pad, mask1, p2_pad,
            x1, wl1_pad, bl1_pad, wl3_pad, bl3_pad):
    return pl.pallas_call(
        _block3_body,
        out_shape=jax.ShapeDtypeStruct((NUM_GRAPHS, 128), jnp.float32),
    )(agg2, g_arr, w2root_pad, b2_pad, batch_pad, mask1, p2_pad,
      x1, wl1_pad, bl1_pad, wl3_pad, bl3_pad)


def _pad2(a, r, c):
    return jnp.pad(a, ((0, r - a.shape[0]), (0, c - a.shape[1])))


def kernel(x, edge_index, batch, emb, W1r, b1r, W1root, p1, W2r, b2r,
           W2root, p2, Wl1, bl1, Wl3, bl3):
    idx = x.astype(jnp.int32)
    idx_pad = jnp.pad(idx, ((0, NP - N), (0, 0)), constant_values=VP - 1)
    batch_pad = jnp.pad(batch.astype(jnp.int32)[:, None], ((0, NP - N), (0, 0)),
                        constant_values=NUM_GRAPHS)

    emb_pad = _pad2(emb, VP - 1, DP)
    emb_pad = jnp.pad(emb_pad, ((0, 1), (0, 0)))
    w1r_pad = _pad2(W1r, DP, DP)
    w1root_pad = _pad2(W1root, DP, DP)
    w2r_pad = _pad2(W2r, DP, DP)
    w2root_pad = _pad2(W2root, DP, DP)
    b1_pad = jnp.pad(b1r, (0, DP - b1r.shape[0]))[None, :]
    b2_pad = jnp.pad(b2r, (0, DP - b2r.shape[0]))[None, :]
    p1_pad = jnp.pad(p1, (0, DP - p1.shape[0]))[:, None]
    p2_pad = jnp.pad(p2, (0, DP - p2.shape[0]))[:, None]
    wl1_pad = _pad2(Wl1, 64, 2 * DP)
    wl1_pad = wl1_pad.at[:, DP:DP + 117].set(wl1_pad[:, 117:234])
    wl1_pad = wl1_pad.at[:, 117:DP].set(0.0)
    bl1_pad = jnp.pad(bl1, (0, 64 - bl1.shape[0]))[None, :]
    wl3_pad = _pad2(Wl3, 128, 64)
    bl3_pad = jnp.pad(bl3, (0, 128 - bl3.shape[0]))[None, :]

    srcp = jnp.pad(edge_index[0], (0, EP - E)).reshape(EP // CHUNK, 1, CHUNK)
    dstp = jnp.pad(edge_index[1], (0, EP - E),
                   constant_values=NP - 1).reshape(EP // CHUNK, 1, CHUNK)
    idx2d = jnp.concatenate([srcp[:, :, None, :], dstp[:, :, None, :]],
                            axis=1)
    zeros = jnp.zeros((NAGG, DP), jnp.float32)

    u1, root1 = _prep(idx_pad, emb_pad, w1r_pad, w1root_pad)
    agg1 = _sc_segsum(u1, idx2d, zeros)
    g_arr, x1, mask1 = _block2(agg1, root1, b1_pad, batch_pad, p1_pad)
    y2 = _proj(g_arr, w2r_pad)
    agg2 = _sc_segsum(y2, idx2d, zeros)
    out = _block3(agg2, g_arr, w2root_pad, b2_pad, batch_pad, mask1, p2_pad,
                  x1, wl1_pad, bl1_pad, wl3_pad, bl3_pad)
    return out[:, 0]


# speedup vs baseline: 1.1004x; 1.0120x over previous
import jax
import jax.numpy as jnp
from jax import lax
from jax.experimental import pallas as pl
from jax.experimental.pallas import tpu as pltpu
from jax.experimental.pallas import tpu_sc as plsc

NUM_GRAPHS = 64
N = 10000
NP = 10240
NAGG = 10368
E = 640000
EP = 647168
CHUNK = 128
CHUNKS_PER_WORKER = EP // (32 * CHUNK)
CH0 = 280
CH1 = 36
DP = 128
VP = 512
IMIN = -2147483648
IMAX = 2147483647


def _segsum_body(vals_hbm, idx_hbm, zeros_hbm, out_hbm,
                 cb0, cb1, rows0, rows1, agg_sh, sem0, sem1):
    c = lax.axis_index("c")
    s = lax.axis_index("s")

    rows_per_tile = NAGG // 16
    pltpu.sync_copy(zeros_hbm.at[pl.ds(s * rows_per_tile, rows_per_tile)],
                    agg_sh.at[pl.ds(s * rows_per_tile, rows_per_tile)])
    plsc.subcore_barrier()

    base = jnp.where(c == 0, s * CH0, 16 * CH0 + s * CH1)
    npairs = jnp.where(c == 0, CH0 // 2, CH1 // 2)

    pltpu.sync_copy(idx_hbm.at[base], cb0)
    pltpu.sync_copy(idx_hbm.at[base + 1], cb1)
    pltpu.async_copy(vals_hbm.at[cb0.at[0, 0]], rows0, sem0)
    pltpu.async_copy(vals_hbm.at[cb1.at[0, 0]], rows1, sem1)

    def step(p, carry):
        j0 = base + 2 * p
        pltpu.make_async_copy(vals_hbm.at[cb0.at[0, 0]], rows0, sem0).wait()
        pltpu.sync_copy(rows0, agg_sh.at[cb0.at[1, 0]], add=True)

        @pl.when(p + 1 < npairs)
        def _():
            pltpu.sync_copy(idx_hbm.at[j0 + 2], cb0)
            pltpu.async_copy(vals_hbm.at[cb0.at[0, 0]], rows0, sem0)

        pltpu.make_async_copy(vals_hbm.at[cb1.at[0, 0]], rows1, sem1).wait()
        pltpu.sync_copy(rows1, agg_sh.at[cb1.at[1, 0]], add=True)

        @pl.when(p + 1 < npairs)
        def _():
            pltpu.sync_copy(idx_hbm.at[j0 + 3], cb1)
            pltpu.async_copy(vals_hbm.at[cb1.at[0, 0]], rows1, sem1)

        return carry

    lax.fori_loop(0, npairs, step, 0)
    plsc.subcore_barrier()

    out_rows = NP // 16
    pltpu.sync_copy(agg_sh.at[pl.ds(s * out_rows, out_rows)],
                    out_hbm.at[c, pl.ds(s * out_rows, out_rows)])


def _sc_segsum(vals, idx2d, zeros):
    mesh = plsc.VectorSubcoreMesh(core_axis_name="c", subcore_axis_name="s")
    f = pl.kernel(
        _segsum_body,
        out_type=jax.ShapeDtypeStruct((2, NP, DP), jnp.float32),
        mesh=mesh,
        scratch_types=[
            pltpu.VMEM((2, 1, CHUNK), jnp.int32),
            pltpu.VMEM((2, 1, CHUNK), jnp.int32),
            pltpu.VMEM((CHUNK, DP), jnp.float32),
            pltpu.VMEM((CHUNK, DP), jnp.float32),
            pltpu.VMEM_SHARED((NAGG, DP), jnp.float32),
            pltpu.SemaphoreType.DMA,
            pltpu.SemaphoreType.DMA,
        ],
    )
    return f(vals, idx2d, zeros)


def _prep_body(idx_ref, emb_ref, wr_ref, wroot_ref, u1_ref, root1_ref):
    emb_v = emb_ref[...]
    tab_rel = lax.dot_general(emb_v, wr_ref[...], (((1,), (1,)), ((), ())),
                              preferred_element_type=jnp.float32)
    tab_root = lax.dot_general(emb_v, wroot_ref[...], (((1,), (1,)), ((), ())),
                               preferred_element_type=jnp.float32)
    idx = idx_ref[...]
    oh = (idx == lax.broadcasted_iota(jnp.int32, (1, VP), 1))
    ohf = oh.astype(jnp.float32)
    u1_ref[...] = jnp.dot(ohf, tab_rel, preferred_element_type=jnp.float32)
    root1_ref[...] = jnp.dot(ohf, tab_root, preferred_element_type=jnp.float32)


def _prep(idx_pad, emb_pad, w1r_pad, w1root_pad):
    return pl.pallas_call(
        _prep_body,
        out_shape=(jax.ShapeDtypeStruct((NP, DP), jnp.float32),
                   jax.ShapeDtypeStruct((NP, DP), jnp.float32)),
    )(idx_pad, emb_pad, w1r_pad, w1root_pad)


def _topk_mask(score_col, batch_col, alive_col):
    bits = lax.bitcast_convert_type(score_col, jnp.int32)
    ki = jnp.where(bits >= 0, bits, jnp.int32(IMIN) - bits)
    ki = jnp.where(alive_col, ki, jnp.int32(IMIN))

    giota = lax.broadcasted_iota(jnp.int32, (1, NUM_GRAPHS), 1)
    galive = (batch_col == giota) & alive_col

    counts = jnp.sum(galive.astype(jnp.float32), axis=0, keepdims=True)
    k = jnp.ceil(jnp.float32(0.8) * counts).astype(jnp.int32)

    def bs_step(_, lohi):
        lo, hi = lohi
        mid = lo + lax.shift_right_logical(hi - lo, 1)
        cnt = jnp.sum((galive & (ki >= mid)).astype(jnp.int32),
                      axis=0, keepdims=True)
        ge = cnt >= k
        return (jnp.where(ge, mid, lo), jnp.where(ge, hi, mid))

    lo0 = jnp.full((1, NUM_GRAPHS), IMIN, jnp.int32)
    hi0 = jnp.full((1, NUM_GRAPHS), IMAX, jnp.int32)
    thr, _ = lax.fori_loop(0, 32, bs_step, (lo0, hi0))

    gt = galive & (ki > thr)
    cnt_gt = jnp.sum(gt.astype(jnp.int32), axis=0, keepdims=True)
    t_budget = k - cnt_gt
    tie = galive & (ki == thr)
    niota = lax.broadcasted_iota(jnp.int32, (NP, 1), 0)

    def idx_step(_, lohi):
        lo, hi = lohi
        mid = lo + lax.shift_right_logical(hi - lo, 1)
        cnt = jnp.sum((tie & (niota < mid)).astype(jnp.int32),
                      axis=0, keepdims=True)
        ge = cnt >= t_budget
        return (jnp.where(ge, lo, mid), jnp.where(ge, mid, hi))

    lo0i = jnp.zeros((1, NUM_GRAPHS), jnp.int32)
    hi0i = jnp.full((1, NUM_GRAPHS), 16384, jnp.int32)
    _, cut = lax.fori_loop(0, 14, idx_step, (lo0i, hi0i))

    keepMT = gt | (tie & (niota < cut) & (t_budget > 0))
    mask_col = jnp.sum(keepMT.astype(jnp.float32), axis=1, keepdims=True)
    return mask_col, keepMT, counts


def _pool_feats(x_new, keepMT):
    keepf = keepMT.astype(jnp.float32)
    counts_new = jnp.sum(keepf, axis=0, keepdims=True)
    sums = lax.dot_general(keepf, x_new, (((0,), (0,)), ((), ())),
                           preferred_element_type=jnp.float32)
    gap = sums / jnp.maximum(counts_new.reshape(NUM_GRAPHS, 1), 1.0)

    giota = lax.broadcasted_iota(jnp.int32, (1, NUM_GRAPHS), 1)
    giota_col = lax.broadcasted_iota(jnp.int32, (NUM_GRAPHS, 1), 0)

    def gmax(g, acc):
        colmask = jnp.sum(jnp.where(giota == g, keepf, 0.0),
                          axis=1, keepdims=True)
        big = jnp.where(colmask > 0, x_new, -jnp.inf)
        m = jnp.max(big, axis=0, keepdims=True)
        return jnp.where(giota_col == g, m, acc)

    gmp = lax.fori_loop(0, NUM_GRAPHS, gmax,
                        jnp.zeros((NUM_GRAPHS, DP), jnp.float32))
    gmp = jnp.where(gmp == -jnp.inf, 0.0, gmp)
    return jnp.concatenate([gmp, gap], axis=1)


def _block2_body(agg_ref, root1_ref, b1_ref, batch_ref, p1_ref,
                 g_ref, x1_ref, mask1_ref):
    agg = agg_ref[0] + agg_ref[1]
    h1 = jnp.maximum(agg + b1_ref[...] + root1_ref[...], 0.0)
    p1 = p1_ref[...]
    score = jnp.dot(h1, p1, preferred_element_type=jnp.float32)
    score = score * lax.rsqrt(jnp.sum(p1 * p1))
    batch = batch_ref[...]
    alive = batch < NUM_GRAPHS
    mask, keepMT, _ = _topk_mask(score, batch, alive)
    g_arr = h1 * (jnp.tanh(score) * mask)
    x1_ref[...] = _pool_feats(g_arr, keepMT)
    g_ref[...] = g_arr
    mask1_ref[...] = mask


def _block2(agg1, root1, b1_pad, batch_pad, p1_pad):
    return pl.pallas_call(
        _block2_body,
        out_shape=(jax.ShapeDtypeStruct((NP, DP), jnp.float32),
                   jax.ShapeDtypeStruct((NUM_GRAPHS, 2 * DP), jnp.float32),
                   jax.ShapeDtypeStruct((NP, 1), jnp.float32)),
    )(agg1, root1, b1_pad, batch_pad, p1_pad)


def _proj_body(g_ref, w_ref, y_ref):
    y_ref[...] = lax.dot_general(g_ref[...], w_ref[...], (((1,), (1,)), ((), ())),
                                 preferred_element_type=jnp.float32)


def _proj(g_arr, w_pad):
    return pl.pallas_call(
        _proj_body,
        out_shape=jax.ShapeDtypeStruct((NP, DP), jnp.float32),
    )(g_arr, w_pad)


def _block3_body(agg_ref, g_ref, w2root_ref, b2_ref, batch_ref, mask1_ref,
                 p2_ref, x1_ref, wl1_ref, bl1_ref, wl3_ref, bl3_ref, out_ref):
    agg = agg_ref[0] + agg_ref[1]
    mask1 = mask1_ref[...]
    root2 = lax.dot_general(g_ref[...], w2root_ref[...],
                            (((1,), (1,)), ((), ())),
                            preferred_element_type=jnp.float32)
    h2 = jnp.maximum(agg + b2_ref[...] + root2, 0.0) * mask1
    p2 = p2_ref[...]
    score = jnp.dot(h2, p2, preferred_element_type=jnp.float32)
    score = score * lax.rsqrt(jnp.sum(p2 * p2))
    batch = batch_ref[...]
    alive = mask1 > 0
    mask2, keepMT, _ = _topk_mask(score, batch, alive)
    x_new = h2 * (jnp.tanh(score) * mask2)
    x2 = _pool_feats(x_new, keepMT)
    feats = x1_ref[...] + x2
    hid = lax.dot_general(feats, wl1_ref[...], (((1,), (1,)), ((), ())),
                          preferred_element_type=jnp.float32)
    hid = jnp.maximum(hid + bl1_ref[...], 0.0)
    o = lax.dot_general(hid, wl3_ref[...], (((1,), (1,)), ((), ())),
                        preferred_element_type=jnp.float32) + bl3_ref[...]
    out_ref[...] = 1.0 / (1.0 + jnp.exp(-o))


def _block3(agg2, g_arr, w2root_pad, b2_pad, batch_pad, mask1, p2_pad,
            x1, wl1_pad, bl1_pad, wl3_pad, bl3_pad):
    return pl.pallas_call(
        _block3_body,
        out_shape=jax.ShapeDtypeStruct((NUM_GRAPHS, 128), jnp.float32),
    )(agg2, g_arr, w2root_pad, b2_pad, batch_pad, mask1, p2_pad,
      x1, wl1_pad, bl1_pad, wl3_pad, bl3_pad)


def _pad2(a, r, c):
    return jnp.pad(a, ((0, r - a.shape[0]), (0, c - a.shape[1])))


def kernel(x, edge_index, batch, emb, W1r, b1r, W1root, p1, W2r, b2r,
           W2root, p2, Wl1, bl1, Wl3, bl3):
    idx = x.astype(jnp.int32)
    idx_pad = jnp.pad(idx, ((0, NP - N), (0, 0)), constant_values=VP - 1)
    batch_pad = jnp.pad(batch.astype(jnp.int32)[:, None], ((0, NP - N), (0, 0)),
                        constant_values=NUM_GRAPHS)

    emb_pad = _pad2(emb, VP - 1, DP)
    emb_pad = jnp.pad(emb_pad, ((0, 1), (0, 0)))
    w1r_pad = _pad2(W1r, DP, DP)
    w1root_pad = _pad2(W1root, DP, DP)
    w2r_pad = _pad2(W2r, DP, DP)
    w2root_pad = _pad2(W2root, DP, DP)
    b1_pad = jnp.pad(b1r, (0, DP - b1r.shape[0]))[None, :]
    b2_pad = jnp.pad(b2r, (0, DP - b2r.shape[0]))[None, :]
    p1_pad = jnp.pad(p1, (0, DP - p1.shape[0]))[:, None]
    p2_pad = jnp.pad(p2, (0, DP - p2.shape[0]))[:, None]
    wl1_pad = _pad2(Wl1, 64, 2 * DP)
    wl1_pad = wl1_pad.at[:, DP:DP + 117].set(wl1_pad[:, 117:234])
    wl1_pad = wl1_pad.at[:, 117:DP].set(0.0)
    bl1_pad = jnp.pad(bl1, (0, 64 - bl1.shape[0]))[None, :]
    wl3_pad = _pad2(Wl3, 128, 64)
    bl3_pad = jnp.pad(bl3, (0, 128 - bl3.shape[0]))[None, :]

    srcp = jnp.pad(edge_index[0], (0, EP - E)).reshape(EP // CHUNK, 1, CHUNK)
    dstp = jnp.pad(edge_index[1], (0, EP - E),
                   constant_values=NP - 1).reshape(EP // CHUNK, 1, CHUNK)
    idx2d = jnp.concatenate([srcp[:, :, None, :], dstp[:, :, None, :]],
                            axis=1)
    zeros = jnp.zeros((NAGG, DP), jnp.float32)

    u1, root1 = _prep(idx_pad, emb_pad, w1r_pad, w1root_pad)
    agg1 = _sc_segsum(u1, idx2d, zeros)
    g_arr, x1, mask1 = _block2(agg1, root1, b1_pad, batch_pad, p1_pad)
    y2 = _proj(g_arr, w2r_pad)
    agg2 = _sc_segsum(y2, idx2d, zeros)
    out = _block3(agg2, g_arr, w2root_pad, b2_pad, batch_pad, mask1, p2_pad,
                  x1, wl1_pad, bl1_pad, wl3_pad, bl3_pad)
    return out[:, 0]
